# Initial kernel scaffold; baseline (speedup 1.0000x reference)
import jax
import jax.numpy as jnp
from jax.experimental import pallas as pl


def kernel(x, edge_index, batch, W_root1, W_nbr1, b1, a1, gamma1, beta1, W_root2, W_nbr2, b2, a2, gamma2, beta2, W_root3, W_nbr3, b3, a3, gamma3, beta3, W_root4, W_nbr4, b4, a4, gamma4, beta4, Wh1, bh1, Wh2, bh2):
    raise NotImplementedError("write your pallas kernel here")


# speedup vs baseline: 2.6223x; 2.6223x over previous
import jax
import jax.numpy as jnp
from jax import lax
from jax.experimental import pallas as pl
from jax.experimental.pallas import tpu as pltpu
from jax.experimental.pallas import tpu_sc as plsc

N = 10000
E = 320000
G = 64
DH = 256

NS = 16
CH = 128
EPT = 20480
E_PAD = NS * EPT
NCHUNK = EPT // CH
N_PAD = 10112
ZROWS = N_PAD // NS

BN = 1000
NB = N // BN


def _r16(v):
    return v.astype(jnp.bfloat16).astype(jnp.float32)


def _r16_strict(v):
    return lax.optimization_barrier(v.astype(jnp.bfloat16)).astype(jnp.float32)


def _make_sc_agg(mode):
    mesh = plsc.VectorSubcoreMesh(core_axis_name="c", subcore_axis_name="s",
                                  num_cores=2, num_subcores=NS)
    ept = EPT if mode == "feat" else EPT // 2
    nchunk = ept // CH

    def body(x2, src2, dst, zeros, out, srcv, dstv, rows, acc, sem):
        c = lax.axis_index("c")
        s = lax.axis_index("s")
        pltpu.sync_copy(zeros.at[pl.ds(s * ZROWS, ZROWS)],
                        acc.at[pl.ds(s * ZROWS, ZROWS)])
        plsc.subcore_barrier()

        if mode == "feat":
            eb = c * E_PAD + s * ept
            db = s * ept
        else:
            eb = (c * NS + s) * ept
            db = eb

        def step(k, carry):
            off = pl.multiple_of(k * CH, CH)
            pltpu.sync_copy(src2.at[pl.ds(eb + off, CH)], srcv)
            pltpu.sync_copy(dst.at[pl.ds(db + off, CH)], dstv)
            pltpu.async_copy(x2.at[srcv], rows, sem).wait()
            pltpu.sync_copy(rows, acc.at[dstv], add=True)
            return carry

        lax.fori_loop(0, nchunk, step, 0)
        plsc.subcore_barrier()
        pltpu.sync_copy(acc.at[pl.ds(s * ZROWS, ZROWS)],
                        out.at[pl.ds(c * N_PAD + s * ZROWS, ZROWS)])

    return pl.kernel(
        body,
        out_type=jax.ShapeDtypeStruct((2 * N_PAD, 128), jnp.float32),
        mesh=mesh,
        scratch_types=[
            pltpu.VMEM((CH,), jnp.int32),
            pltpu.VMEM((CH,), jnp.int32),
            pltpu.VMEM((CH, 128), jnp.float32),
            pltpu.VMEM_SHARED((N_PAD, 128), jnp.float32),
            pltpu.SemaphoreType.DMA,
        ],
    )


def _make_k1(ph, dch, pa, dca, din, agg_sum=False):
    def body(h_ref, a_ref, wr_ref, wn_ref, b_ref, al_ref, act_ref, st_ref,
             ssum, ssq):
        i = pl.program_id(0)
        h = jnp.concatenate([h_ref[p] for p in range(ph)], axis=-1)
        if agg_sum:
            ag = a_ref[0] + a_ref[1]
        else:
            ag = jnp.concatenate([a_ref[p] for p in range(pa)], axis=-1)
        act = (jnp.dot(_r16(h), _r16(wr_ref[...]),
                       preferred_element_type=jnp.float32,
                       precision=lax.Precision.DEFAULT)
               + jnp.dot(ag, _r16(wn_ref[...]),
                         preferred_element_type=jnp.float32,
                         precision=lax.Precision.HIGHEST)
               + b_ref[...])
        act = jnp.where(act >= 0.0, act, al_ref[...] * act)
        act_ref[...] = act

        @pl.when(i == 0)
        def _():
            ssum[...] = jnp.zeros_like(ssum)
            ssq[...] = jnp.zeros_like(ssq)

        ssum[0:1, :] += jnp.sum(act, axis=0, keepdims=True)
        ssq[0:1, :] += jnp.sum(act * act, axis=0, keepdims=True)
        st_ref[0:1, :] = ssum[0:1, :]
        st_ref[1:2, :] = ssq[0:1, :]

    return pl.pallas_call(
        body,
        grid=(NB,),
        in_specs=[
            pl.BlockSpec((ph, BN, dch), lambda i: (0, i, 0)),
            pl.BlockSpec((pa, BN, dca), lambda i: (0, i, 0)),
            pl.BlockSpec((din, DH), lambda i: (0, 0)),
            pl.BlockSpec((din, DH), lambda i: (0, 0)),
            pl.BlockSpec((1, DH), lambda i: (0, 0)),
            pl.BlockSpec((1, DH), lambda i: (0, 0)),
        ],
        out_specs=[
            pl.BlockSpec((BN, DH), lambda i: (i, 0)),
            pl.BlockSpec((8, DH), lambda i: (0, 0)),
        ],
        out_shape=[
            jax.ShapeDtypeStruct((N, DH), jnp.float32),
            jax.ShapeDtypeStruct((8, DH), jnp.float32),
        ],
        scratch_shapes=[
            pltpu.VMEM((8, DH), jnp.float32),
            pltpu.VMEM((8, DH), jnp.float32),
        ],
    )


def _make_k2():
    def body(act_ref, st_ref, g_ref, be_ref, out_ref):
        mean = st_ref[0:1, :] * (1.0 / N)
        ex2 = st_ref[1:2, :] * (1.0 / N)
        var = ex2 - mean * mean
        scale = g_ref[...] * lax.rsqrt(var + 1e-5)
        shift = be_ref[...] - mean * scale
        hcur = act_ref[...] * scale + shift
        out_ref[0, :, :] = hcur[:, :128]
        out_ref[1, :, :] = hcur[:, 128:]

    return pl.pallas_call(
        body,
        grid=(NB,),
        in_specs=[
            pl.BlockSpec((BN, DH), lambda i: (i, 0)),
            pl.BlockSpec((8, DH), lambda i: (0, 0)),
            pl.BlockSpec((1, DH), lambda i: (0, 0)),
            pl.BlockSpec((1, DH), lambda i: (0, 0)),
        ],
        out_specs=pl.BlockSpec((2, BN, 128), lambda i: (0, i, 0)),
        out_shape=jax.ShapeDtypeStruct((2, N, 128), jnp.float32),
    )


def _make_pool():
    def body(h_ref, b_ref, w1_ref, b1_ref, w2_ref, b2_ref, o_ref, pacc, cacc):
        i = pl.program_id(0)

        @pl.when(i == 0)
        def _():
            pacc[...] = jnp.zeros_like(pacc)
            cacc[...] = jnp.zeros_like(cacc)

        h = jnp.concatenate([h_ref[0], h_ref[1]], axis=-1)
        ids = b_ref[0]
        rows = lax.broadcasted_iota(jnp.int32, (G, BN), 0)
        oneT = (rows == ids).astype(jnp.float32)
        pacc[...] += jnp.dot(oneT, h, preferred_element_type=jnp.float32, precision=lax.Precision.HIGHEST)
        cacc[...] += jnp.dot(oneT, jnp.ones((BN, DH), jnp.float32),
                             preferred_element_type=jnp.float32, precision=lax.Precision.HIGHEST)

        @pl.when(i == NB - 1)
        def _():
            pooled = pacc[...] / jnp.maximum(cacc[...], 1.0)
            hid = jnp.maximum(
                jnp.dot(_r16(pooled), _r16(w1_ref[...]),
                        preferred_element_type=jnp.float32,
                        precision=lax.Precision.DEFAULT)
                + b1_ref[...], 0.0)
            o_ref[...] = (jnp.dot(_r16(hid), _r16(w2_ref[...]),
                                  preferred_element_type=jnp.float32,
                                  precision=lax.Precision.DEFAULT)
                          + b2_ref[...])

    return pl.pallas_call(
        body,
        grid=(NB,),
        in_specs=[
            pl.BlockSpec((2, BN, 128), lambda i: (0, i, 0)),
            pl.BlockSpec((1, 1, BN), lambda i: (i, 0, 0)),
            pl.BlockSpec((DH, DH), lambda i: (0, 0)),
            pl.BlockSpec((1, DH), lambda i: (0, 0)),
            pl.BlockSpec((DH, 128), lambda i: (0, 0)),
            pl.BlockSpec((1, 128), lambda i: (0, 0)),
        ],
        out_specs=pl.BlockSpec((G, 128), lambda i: (0, 0)),
        out_shape=jax.ShapeDtypeStruct((G, 128), jnp.float32),
        scratch_shapes=[
            pltpu.VMEM((G, DH), jnp.float32),
            pltpu.VMEM((G, DH), jnp.float32),
        ],
    )


_sc_cache = {}


def _sc_agg(dc):
    if dc not in _sc_cache:
        _sc_cache[dc] = _make_sc_agg(dc)
    return _sc_cache[dc]


_k1_first = _make_k1(1, 128, 2, 128, 128, agg_sum=True)
_k1_hidden = _make_k1(2, 128, 2, 128, DH)
_k2 = _make_k2()
_pool = _make_pool()


def kernel(x, edge_index, batch,
           W_root1, W_nbr1, b1, a1, gamma1, beta1,
           W_root2, W_nbr2, b2, a2, gamma2, beta2,
           W_root3, W_nbr3, b3, a3, gamma3, beta3,
           W_root4, W_nbr4, b4, a4, gamma4, beta4,
           Wh1, bh1, Wh2, bh2):
    src = edge_index[0].astype(jnp.int32)
    dst = edge_index[1].astype(jnp.int32)
    pad = E_PAD - E
    srcp = jnp.concatenate([src, jnp.zeros((pad,), jnp.int32)])
    src2 = jnp.concatenate([srcp, srcp + N])
    dstp = jnp.concatenate([dst, jnp.full((pad,), N, jnp.int32)])
    zeros128 = jnp.zeros((N_PAD, 128), jnp.float32)
    batch3 = batch.astype(jnp.int32).reshape(NB, 1, BN)

    def row2(v):
        return v.reshape(1, DH)

    def slope(a):
        return jnp.broadcast_to(a.reshape(1, 1), (1, DH))

    agg = _sc_agg("edge")(_r16_strict(x), srcp, dstp,
                          zeros128).reshape(2, N_PAD, 128)[:, :N, :]
    act, st = _k1_first(x.reshape(1, N, 128), agg, W_root1, W_nbr1,
                        row2(b1), slope(a1))
    hs = _k2(act, st, row2(gamma1), row2(beta1))

    for (Wr, Wn, b, a, gm, bt) in (
            (W_root2, W_nbr2, b2, a2, gamma2, beta2),
            (W_root3, W_nbr3, b3, a3, gamma3, beta3),
            (W_root4, W_nbr4, b4, a4, gamma4, beta4)):
        agg = _sc_agg("feat")(_r16_strict(hs).reshape(2 * N, 128), src2, dstp,
                              zeros128).reshape(2, N_PAD, 128)[:, :N, :]
        act, st = _k1_hidden(hs, agg, Wr, Wn, row2(b), slope(a))
        hs = _k2(act, st, row2(gm), row2(bt))

    Wh2p = jnp.pad(Wh2, ((0, 0), (0, 127)))
    bh2p = jnp.broadcast_to(bh2.reshape(1, 1), (1, 128))
    out = _pool(hs, batch3, Wh1, bh1.reshape(1, DH), Wh2p, bh2p)
    return out[:, :1]


# speedup vs baseline: 3.4329x; 1.3091x over previous
import jax
import jax.numpy as jnp
from jax import lax
from jax.experimental import pallas as pl
from jax.experimental.pallas import tpu as pltpu
from jax.experimental.pallas import tpu_sc as plsc

N = 10000
E = 320000
G = 64
DH = 256

NS = 16
CH = 128
EPT = 20480
E_PAD = NS * EPT
NCHUNK = EPT // CH
N_PAD = 10112
ZROWS = N_PAD // NS

BN = 1000
NB = N // BN


def _r16(v):
    return v.astype(jnp.bfloat16).astype(jnp.float32)


def _r16_strict(v):
    return lax.optimization_barrier(v.astype(jnp.bfloat16)).astype(jnp.float32)


def _make_sc_agg(mode):
    mesh = plsc.VectorSubcoreMesh(core_axis_name="c", subcore_axis_name="s",
                                  num_cores=2, num_subcores=NS)
    ept = EPT if mode == "feat" else EPT // 2
    nchunk = ept // CH
    BLK = 8
    nblk = nchunk // BLK
    nblk2 = nblk // 2

    def body(x2, src3, dst3, zeros, out, srcb, dstb, rows, acc,
             semg0, semg1, sems0, sems1, semi0, semi1):
        c = lax.axis_index("c")
        s = lax.axis_index("s")
        w = c * NS + s
        semg = (semg0, semg1)
        sems = (sems0, sems1)
        semi = (semi0, semi1)
        drow = s if mode == "feat" else w

        pltpu.sync_copy(zeros.at[pl.ds(s * ZROWS, ZROWS)],
                        acc.at[pl.ds(s * ZROWS, ZROWS)])
        pltpu.sync_copy(src3.at[w, pl.ds(0, BLK)], srcb.at[0])
        pltpu.sync_copy(dst3.at[drow, pl.ds(0, BLK)], dstb.at[0])
        plsc.subcore_barrier()

        def istart(j, p):
            off = pl.multiple_of(j * BLK, BLK)
            pltpu.async_copy(src3.at[w, pl.ds(off, BLK)], srcb.at[p],
                             semi[p])
            pltpu.async_copy(dst3.at[drow, pl.ds(off, BLK)], dstb.at[p],
                             semi[p])

        def iwait(p):
            pltpu.make_async_copy(src3.at[w, pl.ds(0, BLK)], srcb.at[p],
                                  semi[p]).wait()
            pltpu.make_async_copy(dst3.at[drow, pl.ds(0, BLK)], dstb.at[p],
                                  semi[p]).wait()

        def g_start(p, k, b):
            pltpu.async_copy(x2.at[srcb.at[p, k]], rows.at[b], semg[b])

        def g_wait(b):
            pltpu.make_async_copy(x2.at[srcb.at[0, 0]], rows.at[b],
                                  semg[b]).wait()

        def s_start(p, k, b):
            pltpu.async_copy(rows.at[b], acc.at[dstb.at[p, k]], sems[b],
                             add=True)

        def s_wait(b):
            pltpu.make_async_copy(rows.at[b], acc.at[dstb.at[0, 0]],
                                  sems[b]).wait()

        def do_block(p):
            g_start(p, 0, 0)
            g_start(p, 1, 1)
            for k in range(BLK):
                b = k % 2
                g_wait(b)
                s_start(p, k, b)
                if k + 2 < BLK:
                    s_wait(b)
                    g_start(p, k + 2, b)
            s_wait(0)
            s_wait(1)

        def blk_step(j2, carry):
            j = j2 * 2
            istart(j + 1, 1)
            do_block(0)

            @pl.when(j2 + 1 < nblk2)
            def _():
                istart(j + 2, 0)

            iwait(1)
            do_block(1)

            @pl.when(j2 + 1 < nblk2)
            def _():
                iwait(0)

            return carry

        lax.fori_loop(0, nblk2, blk_step, 0)
        plsc.subcore_barrier()
        pltpu.sync_copy(acc.at[pl.ds(s * ZROWS, ZROWS)],
                        out.at[pl.ds(c * N_PAD + s * ZROWS, ZROWS)])

    return pl.kernel(
        body,
        out_type=jax.ShapeDtypeStruct((2 * N_PAD, 128), jnp.float32),
        mesh=mesh,
        scratch_types=[
            pltpu.VMEM((2, BLK, CH), jnp.int32),
            pltpu.VMEM((2, BLK, CH), jnp.int32),
            pltpu.VMEM((2, CH, 128), jnp.float32),
            pltpu.VMEM_SHARED((N_PAD, 128), jnp.float32),
            pltpu.SemaphoreType.DMA,
            pltpu.SemaphoreType.DMA,
            pltpu.SemaphoreType.DMA,
            pltpu.SemaphoreType.DMA,
            pltpu.SemaphoreType.DMA,
            pltpu.SemaphoreType.DMA,
        ],
    )


def _make_k1(ph, dch, pa, dca, din, agg_sum=False):
    def body(h_ref, a_ref, wr_ref, wn_ref, b_ref, al_ref, act_ref, st_ref,
             ssum, ssq):
        i = pl.program_id(0)
        h = jnp.concatenate([h_ref[p] for p in range(ph)], axis=-1)
        if agg_sum:
            ag = a_ref[0] + a_ref[1]
        else:
            ag = jnp.concatenate([a_ref[p] for p in range(pa)], axis=-1)
        act = (jnp.dot(_r16(h), _r16(wr_ref[...]),
                       preferred_element_type=jnp.float32,
                       precision=lax.Precision.DEFAULT)
               + jnp.dot(ag, _r16(wn_ref[...]),
                         preferred_element_type=jnp.float32,
                         precision=lax.Precision.HIGHEST)
               + b_ref[...])
        act = jnp.where(act >= 0.0, act, al_ref[...] * act)
        act_ref[...] = act

        @pl.when(i == 0)
        def _():
            ssum[...] = jnp.zeros_like(ssum)
            ssq[...] = jnp.zeros_like(ssq)

        ssum[0:1, :] += jnp.sum(act, axis=0, keepdims=True)
        ssq[0:1, :] += jnp.sum(act * act, axis=0, keepdims=True)
        st_ref[0:1, :] = ssum[0:1, :]
        st_ref[1:2, :] = ssq[0:1, :]

    return pl.pallas_call(
        body,
        grid=(NB,),
        in_specs=[
            pl.BlockSpec((ph, BN, dch), lambda i: (0, i, 0)),
            pl.BlockSpec((pa, BN, dca), lambda i: (0, i, 0)),
            pl.BlockSpec((din, DH), lambda i: (0, 0)),
            pl.BlockSpec((din, DH), lambda i: (0, 0)),
            pl.BlockSpec((1, DH), lambda i: (0, 0)),
            pl.BlockSpec((1, DH), lambda i: (0, 0)),
        ],
        out_specs=[
            pl.BlockSpec((BN, DH), lambda i: (i, 0)),
            pl.BlockSpec((8, DH), lambda i: (0, 0)),
        ],
        out_shape=[
            jax.ShapeDtypeStruct((N, DH), jnp.float32),
            jax.ShapeDtypeStruct((8, DH), jnp.float32),
        ],
        scratch_shapes=[
            pltpu.VMEM((8, DH), jnp.float32),
            pltpu.VMEM((8, DH), jnp.float32),
        ],
    )


def _make_k2():
    def body(act_ref, st_ref, g_ref, be_ref, out_ref):
        mean = st_ref[0:1, :] * (1.0 / N)
        ex2 = st_ref[1:2, :] * (1.0 / N)
        var = ex2 - mean * mean
        scale = g_ref[...] * lax.rsqrt(var + 1e-5)
        shift = be_ref[...] - mean * scale
        hcur = act_ref[...] * scale + shift
        out_ref[0, :, :] = hcur[:, :128]
        out_ref[1, :, :] = hcur[:, 128:]

    return pl.pallas_call(
        body,
        grid=(NB,),
        in_specs=[
            pl.BlockSpec((BN, DH), lambda i: (i, 0)),
            pl.BlockSpec((8, DH), lambda i: (0, 0)),
            pl.BlockSpec((1, DH), lambda i: (0, 0)),
            pl.BlockSpec((1, DH), lambda i: (0, 0)),
        ],
        out_specs=pl.BlockSpec((2, BN, 128), lambda i: (0, i, 0)),
        out_shape=jax.ShapeDtypeStruct((2, N, 128), jnp.float32),
    )


def _make_pool():
    def body(h_ref, b_ref, w1_ref, b1_ref, w2_ref, b2_ref, o_ref, pacc, cacc):
        i = pl.program_id(0)

        @pl.when(i == 0)
        def _():
            pacc[...] = jnp.zeros_like(pacc)
            cacc[...] = jnp.zeros_like(cacc)

        h = jnp.concatenate([h_ref[0], h_ref[1]], axis=-1)
        ids = b_ref[0]
        rows = lax.broadcasted_iota(jnp.int32, (G, BN), 0)
        oneT = (rows == ids).astype(jnp.float32)
        pacc[...] += jnp.dot(oneT, h, preferred_element_type=jnp.float32, precision=lax.Precision.HIGHEST)
        cacc[...] += jnp.dot(oneT, jnp.ones((BN, DH), jnp.float32),
                             preferred_element_type=jnp.float32, precision=lax.Precision.HIGHEST)

        @pl.when(i == NB - 1)
        def _():
            pooled = pacc[...] / jnp.maximum(cacc[...], 1.0)
            hid = jnp.maximum(
                jnp.dot(_r16(pooled), _r16(w1_ref[...]),
                        preferred_element_type=jnp.float32,
                        precision=lax.Precision.DEFAULT)
                + b1_ref[...], 0.0)
            o_ref[...] = (jnp.dot(_r16(hid), _r16(w2_ref[...]),
                                  preferred_element_type=jnp.float32,
                                  precision=lax.Precision.DEFAULT)
                          + b2_ref[...])

    return pl.pallas_call(
        body,
        grid=(NB,),
        in_specs=[
            pl.BlockSpec((2, BN, 128), lambda i: (0, i, 0)),
            pl.BlockSpec((1, 1, BN), lambda i: (i, 0, 0)),
            pl.BlockSpec((DH, DH), lambda i: (0, 0)),
            pl.BlockSpec((1, DH), lambda i: (0, 0)),
            pl.BlockSpec((DH, 128), lambda i: (0, 0)),
            pl.BlockSpec((1, 128), lambda i: (0, 0)),
        ],
        out_specs=pl.BlockSpec((G, 128), lambda i: (0, 0)),
        out_shape=jax.ShapeDtypeStruct((G, 128), jnp.float32),
        scratch_shapes=[
            pltpu.VMEM((G, DH), jnp.float32),
            pltpu.VMEM((G, DH), jnp.float32),
        ],
    )


_sc_cache = {}


def _sc_agg(dc):
    if dc not in _sc_cache:
        _sc_cache[dc] = _make_sc_agg(dc)
    return _sc_cache[dc]


_k1_first = _make_k1(1, 128, 2, 128, 128, agg_sum=True)
_k1_hidden = _make_k1(2, 128, 2, 128, DH)
_k2 = _make_k2()
_pool = _make_pool()


def kernel(x, edge_index, batch,
           W_root1, W_nbr1, b1, a1, gamma1, beta1,
           W_root2, W_nbr2, b2, a2, gamma2, beta2,
           W_root3, W_nbr3, b3, a3, gamma3, beta3,
           W_root4, W_nbr4, b4, a4, gamma4, beta4,
           Wh1, bh1, Wh2, bh2):
    src = edge_index[0].astype(jnp.int32)
    dst = edge_index[1].astype(jnp.int32)
    pad = E_PAD - E
    srcp = jnp.concatenate([src, jnp.zeros((pad,), jnp.int32)])
    src2 = jnp.concatenate([srcp, srcp + N])
    dstp = jnp.concatenate([dst, jnp.full((pad,), N, jnp.int32)])
    zeros128 = jnp.zeros((N_PAD, 128), jnp.float32)
    batch3 = batch.astype(jnp.int32).reshape(NB, 1, BN)

    def row2(v):
        return v.reshape(1, DH)

    def slope(a):
        return jnp.broadcast_to(a.reshape(1, 1), (1, DH))

    src3e = srcp.reshape(2 * NS, EPT // 2 // CH, CH)
    dst3e = dstp.reshape(2 * NS, EPT // 2 // CH, CH)
    src3f = src2.reshape(2 * NS, NCHUNK, CH)
    dst3f = dstp.reshape(NS, NCHUNK, CH)

    agg = _sc_agg("edge")(_r16_strict(x), src3e, dst3e,
                          zeros128).reshape(2, N_PAD, 128)[:, :N, :]
    act, st = _k1_first(x.reshape(1, N, 128), agg, W_root1, W_nbr1,
                        row2(b1), slope(a1))
    hs = _k2(act, st, row2(gamma1), row2(beta1))

    for (Wr, Wn, b, a, gm, bt) in (
            (W_root2, W_nbr2, b2, a2, gamma2, beta2),
            (W_root3, W_nbr3, b3, a3, gamma3, beta3),
            (W_root4, W_nbr4, b4, a4, gamma4, beta4)):
        agg = _sc_agg("feat")(_r16_strict(hs).reshape(2 * N, 128), src3f,
                              dst3f,
                              zeros128).reshape(2, N_PAD, 128)[:, :N, :]
        act, st = _k1_hidden(hs, agg, Wr, Wn, row2(b), slope(a))
        hs = _k2(act, st, row2(gm), row2(bt))

    Wh2p = jnp.pad(Wh2, ((0, 0), (0, 127)))
    bh2p = jnp.broadcast_to(bh2.reshape(1, 1), (1, 128))
    out = _pool(hs, batch3, Wh1, bh1.reshape(1, DH), Wh2p, bh2p)
    return out[:, :1]
